# baseline (speedup 1.0000x reference)
import functools

import jax
import jax.numpy as jnp
from jax import lax
from jax.experimental import pallas as pl
from jax.experimental.pallas import tpu as pltpu
from jax.experimental.pallas import tpu_sc as plsc

N = 10000
D = 128
R = 8
E = 320000

NC = 2
NS = 16
NW = NC * NS
L = 16

B = 128
NBLK = -(-E // (NW * B))
EB = NBLK * B
E_PAD = NW * EB

N_PAD = NBLK * B
NRB = N_PAD // B

K0 = 59
K1 = 2 * NBLK - K0
TOT_BLK = NS * 2 * NBLK

_mesh = functools.partial(
    plsc.VectorSubcoreMesh, core_axis_name="c", subcore_axis_name="s")
_SC_PARAMS = pltpu.CompilerParams(needs_layout_passes=False)


def _wid():
  return lax.axis_index("s") * NC + lax.axis_index("c")


def _count_body(dst_hbm, type_hbm, cnt_hbm, dstb, typeb, tbl):
  wid = _wid()

  def zero_step(i, _):
    tbl[pl.ds(i * L, L)] = jnp.zeros((L,), jnp.float32)
    return 0
  lax.fori_loop(0, (N * R) // L, zero_step, 0)

  pltpu.sync_copy(dst_hbm.at[wid], dstb)
  pltpu.sync_copy(type_hbm.at[wid], typeb)

  ones = jnp.ones((L,), jnp.float32)
  iota = lax.iota(jnp.int32, L)
  base0 = wid * EB

  def step(i, _):
    d = dstb[pl.ds(i * L, L)]
    t = typeb[pl.ds(i * L, L)]
    idx = d * R + t
    valid = (base0 + i * L + iota) < E
    plsc.addupdate_scatter(tbl, [idx], ones, mask=valid)
    return 0
  lax.fori_loop(0, EB // L, step, 0)

  pltpu.sync_copy(tbl, cnt_hbm.at[wid])


def _count_phase(dst_pad, type_pad):
  k = pl.kernel(
      _count_body,
      out_type=jax.ShapeDtypeStruct((NW, N * R), jnp.float32),
      mesh=_mesh(),
      compiler_params=_SC_PARAMS,
      scratch_types=[
          pltpu.VMEM((EB,), jnp.int32),
          pltpu.VMEM((EB,), jnp.int32),
          pltpu.VMEM((N * R,), jnp.float32),
      ],
  )
  return k(dst_pad.reshape(NW, EB), type_pad.reshape(NW, EB))


def _wtbl_body(cnt_ref, w_ref):
  s = jnp.sum(cnt_ref[...], axis=0)
  w_ref[...] = 1.0 / jnp.maximum(s, 1.0)


def _wtbl_phase(cnt_parts):
  nrows = (N * R) // 128
  return pl.pallas_call(
      _wtbl_body,
      out_shape=jax.ShapeDtypeStruct((nrows, 128), jnp.float32),
  )(cnt_parts.reshape(NW, nrows, 128))


def _edge_body(src_hbm, dst_hbm, type_hbm, wtbl_hbm,
               hidx_hbm, wedge_hbm, srcb, dstb, typeb, wt):
  wid = _wid()
  pltpu.sync_copy(wtbl_hbm, wt)
  pltpu.sync_copy(src_hbm.at[wid], srcb)
  pltpu.sync_copy(dst_hbm.at[wid], dstb)
  pltpu.sync_copy(type_hbm.at[wid], typeb)

  iota = lax.iota(jnp.int32, L)
  base0 = wid * EB

  def step(i4, _):
    for u in range(4):
      i = i4 * 4 + u
      sl = pl.ds(i * L, L)
      s = srcb[sl]
      t = typeb[sl]
      d = dstb[sl]
      srcb[sl] = t * N + s
      w16 = plsc.load_gather(wt, [d * R + t])
      valid = (base0 + i * L + iota) < E
      w16 = jnp.where(valid, w16, 0.0)
      typeb[sl] = plsc.bitcast(w16, jnp.int32)
    return 0
  lax.fori_loop(0, EB // L // 4, step, 0)

  pltpu.sync_copy(srcb, hidx_hbm.at[wid])
  pltpu.sync_copy(typeb, wedge_hbm.at[wid])


def _edge_phase(src_pad, dst_pad, type_pad, wtbl):
  k = pl.kernel(
      _edge_body,
      out_type=(jax.ShapeDtypeStruct((NW, EB), jnp.int32),
                jax.ShapeDtypeStruct((NW, EB), jnp.int32)),
      mesh=_mesh(),
      compiler_params=_SC_PARAMS,
      scratch_types=[
          pltpu.VMEM((EB,), jnp.int32),
          pltpu.VMEM((EB,), jnp.int32),
          pltpu.VMEM((EB,), jnp.int32),
          pltpu.VMEM((N * R,), jnp.float32),
      ],
  )
  hidx, wedge_i32 = k(src_pad.reshape(NW, EB), dst_pad.reshape(NW, EB),
                      type_pad.reshape(NW, EB), wtbl.reshape(N * R))
  packed = jnp.stack([hidx.reshape(TOT_BLK, B),
                      dst_pad.reshape(TOT_BLK, B),
                      wedge_i32.reshape(TOT_BLK, B)], axis=1)
  return packed


def _msg_body(h_hbm, pk_hbm, out_hbm,
              buf0, buf1, x0, x1, out_sh, rs0, rs1, xs0, xs1):
  c = lax.axis_index("c")
  s = lax.axis_index("s")
  nblk = jnp.where(c == 0, K0, K1)
  base = s * (2 * NBLK) + c * K0
  bufs = (buf0, buf1)
  rsems = (rs0, rs1)
  xbufs = (x0, x1)
  xsems = (xs0, xs1)

  def zbuf(k4, _):
    for u in range(4):
      for cc in range(D // L):
        buf0[k4 * 4 + u, pl.ds(cc * L, L)] = jnp.zeros((L,), jnp.float32)
    return 0
  lax.fori_loop(0, B // 4, zbuf, 0)

  def zrow(g, _):
    b = s + g * NS
    @pl.when(b < NRB)
    def _():
      pltpu.sync_copy(buf0, out_sh.at[pl.ds(b * B, B)])
    return 0
  lax.fori_loop(0, -(-NRB // NS), zrow, 0)
  plsc.subcore_barrier()

  def xload(j, p):
    return pltpu.make_async_copy(pk_hbm.at[base + j], xbufs[p], xsems[p])

  def rgather(j, p):
    return pltpu.make_async_copy(
        h_hbm.at[xbufs[p].at[0]], bufs[p], rsems[p])

  xload(0, 0).start()
  xload(1, 1).start()
  xload(0, 0).wait()
  rgather(0, 0).start()

  def pair(g, _):
    for p in (0, 1):
      j = g * 2 + p

      @pl.when(j < nblk)
      def _():
        rgather(j, p).wait()

        @pl.when(j + 1 < nblk)
        def _():
          xload(j + 1, 1 - p).wait()
          rgather(j + 1, 1 - p).start()

        bufp = bufs[p]
        xp = xbufs[p]

        def scale(k4, _):
          for u in range(4):
            k = k4 * 4 + u
            wbits = plsc.load_gather(xp, [jnp.full((L,), 2, jnp.int32),
                                          jnp.full((L,), k, jnp.int32)])
            w16 = plsc.bitcast(wbits, jnp.float32)
            for cc in range(D // L):
              sl = pl.ds(cc * L, L)
              bufp[k, sl] = bufp[k, sl] * w16
          return 0
        lax.fori_loop(0, B // 4, scale, 0)

        pltpu.sync_copy(bufp, out_sh.at[xp.at[1]], add=True)

        @pl.when(j + 2 < nblk)
        def _():
          xload(j + 2, p).start()
    return 0
  lax.fori_loop(0, (max(K0, K1) + 1) // 2, pair, 0)

  plsc.subcore_barrier()

  def wrow(g, _):
    b = s + g * NS
    @pl.when(b < NRB)
    def _():
      rs = pl.ds(b * B, B)
      pltpu.sync_copy(out_sh.at[rs], out_hbm.at[c, rs])
    return 0
  lax.fori_loop(0, -(-NRB // NS), wrow, 0)


def _msg_phase(h_flat, packed):
  k = pl.kernel(
      _msg_body,
      out_type=jax.ShapeDtypeStruct((NC, N_PAD, D), jnp.float32),
      mesh=_mesh(),
      compiler_params=_SC_PARAMS,
      scratch_types=[
          pltpu.VMEM((B, D), jnp.float32),
          pltpu.VMEM((B, D), jnp.float32),
          pltpu.VMEM((3, B), jnp.int32),
          pltpu.VMEM((3, B), jnp.int32),
          pltpu.VMEM_SHARED((N_PAD, D), jnp.float32),
          pltpu.SemaphoreType.DMA,
          pltpu.SemaphoreType.DMA,
          pltpu.SemaphoreType.DMA,
          pltpu.SemaphoreType.DMA,
      ],
  )
  return k(h_flat, packed)[:, :N, :]


def _dense_phase(xa, xb, w, root, b, relu_in):
  blk = 1000
  nrel = w.shape[0]
  dout = w.shape[2]

  def body(*refs):
    if relu_in:
      xa_ref, xb_ref, w_ref, root_ref, b_ref, h_ref, init_ref = refs
      x = jax.nn.relu(xa_ref[0] + xa_ref[1] + xb_ref[...])
    else:
      xa_ref, w_ref, root_ref, b_ref, h_ref, init_ref = refs
      x = xa_ref[...]
    for r in range(nrel):
      h_ref[r] = jnp.dot(x, w_ref[r], preferred_element_type=jnp.float32)
    init_ref[...] = (
        jnp.dot(x, root_ref[...], preferred_element_type=jnp.float32)
        + b_ref[...])

  if relu_in:
    x_specs = [pl.BlockSpec((NC, blk, D), lambda i: (0, i, 0)),
               pl.BlockSpec((blk, D), lambda i: (i, 0))]
    x_args = [xa, xb]
  else:
    x_specs = [pl.BlockSpec((blk, D), lambda i: (i, 0))]
    x_args = [xa]

  h, init = pl.pallas_call(
      body,
      grid=(N // blk,),
      in_specs=x_specs + [
          pl.BlockSpec((nrel, D, dout), lambda i: (0, 0, 0)),
          pl.BlockSpec((D, dout), lambda i: (0, 0)),
          pl.BlockSpec((1, dout), lambda i: (0, 0)),
      ],
      out_specs=[
          pl.BlockSpec((nrel, blk, dout), lambda i: (0, i, 0)),
          pl.BlockSpec((blk, dout), lambda i: (i, 0)),
      ],
      out_shape=[
          jax.ShapeDtypeStruct((nrel, N, dout), jnp.float32),
          jax.ShapeDtypeStruct((N, dout), jnp.float32),
      ],
  )(*x_args, w, root, b.reshape(1, dout))
  return h.reshape(nrel * N, dout), init


def _final_body(p_ref, init_ref, cw_ref, cb_ref, o_ref):
  h = jax.nn.relu(p_ref[0] + p_ref[1] + init_ref[...])
  o_ref[...] = (
      jnp.dot(h, cw_ref[...], preferred_element_type=jnp.float32)
      + cb_ref[...])


def _final_phase(parts, init, cls_w, cls_b):
  blk = 1000
  out = pl.pallas_call(
      _final_body,
      grid=(N // blk,),
      in_specs=[
          pl.BlockSpec((NC, blk, D), lambda i: (0, i, 0)),
          pl.BlockSpec((blk, D), lambda i: (i, 0)),
          pl.BlockSpec((D, 1), lambda i: (0, 0)),
          pl.BlockSpec((1, 1), lambda i: (0, 0)),
      ],
      out_specs=pl.BlockSpec((blk, 1), lambda i: (i, 0)),
      out_shape=jax.ShapeDtypeStruct((N, 1), jnp.float32),
  )(parts, init, cls_w, cls_b.reshape(1, 1))
  return out.reshape(N)


@jax.jit
def kernel(x, edge_index, edge_type, W1, root1, b1, W2, root2, b2,
           cls_w, cls_b):
  pad = E_PAD - E
  src_pad = jnp.concatenate(
      [edge_index[0], jnp.zeros((pad,), jnp.int32)])
  dst_pad = jnp.concatenate(
      [edge_index[1], jnp.zeros((pad,), jnp.int32)])
  type_pad = jnp.concatenate([edge_type, jnp.zeros((pad,), jnp.int32)])

  cnt_parts = _count_phase(dst_pad, type_pad)
  wtbl = _wtbl_phase(cnt_parts)
  packed = _edge_phase(src_pad, dst_pad, type_pad, wtbl)

  h1, init1 = _dense_phase(x, None, W1, root1, b1, relu_in=False)
  p1 = _msg_phase(h1, packed)
  h2, init2 = _dense_phase(p1, init1, W2, root2, b2, relu_in=True)
  p2 = _msg_phase(h2, packed)
  return _final_phase(p2, init2, cls_w, cls_b)


# speedup vs baseline: 1.2214x; 1.2214x over previous
import functools

import jax
import jax.numpy as jnp
from jax import lax
from jax.experimental import pallas as pl
from jax.experimental.pallas import tpu as pltpu
from jax.experimental.pallas import tpu_sc as plsc

N = 10000
D = 128
R = 8
E = 320000

NC = 2
NS = 16
NW = NC * NS
L = 16

B = 128
NBLK = -(-E // (NW * B))
EB = NBLK * B
E_PAD = NW * EB

N_PAD = NBLK * B
NRB = N_PAD // B

K0 = 99
K1 = 2 * NBLK - K0
TOT_BLK = NS * 2 * NBLK

_mesh = functools.partial(
    plsc.VectorSubcoreMesh, core_axis_name="c", subcore_axis_name="s")
_SC_PARAMS = pltpu.CompilerParams(needs_layout_passes=False)


def _wid():
  return lax.axis_index("s") * NC + lax.axis_index("c")


def _count_body(dst_hbm, type_hbm, cnt_hbm, dstb, typeb, tbl):
  wid = _wid()

  def zero_step(i, _):
    tbl[pl.ds(i * L, L)] = jnp.zeros((L,), jnp.float32)
    return 0
  lax.fori_loop(0, (N * R) // L, zero_step, 0)

  pltpu.sync_copy(dst_hbm.at[wid], dstb)
  pltpu.sync_copy(type_hbm.at[wid], typeb)

  ones = jnp.ones((L,), jnp.float32)
  iota = lax.iota(jnp.int32, L)
  base0 = wid * EB

  def step(i, _):
    d = dstb[pl.ds(i * L, L)]
    t = typeb[pl.ds(i * L, L)]
    idx = d * R + t
    valid = (base0 + i * L + iota) < E
    plsc.addupdate_scatter(tbl, [idx], ones, mask=valid)
    return 0
  lax.fori_loop(0, EB // L, step, 0)

  pltpu.sync_copy(tbl, cnt_hbm.at[wid])


def _count_phase(dst_pad, type_pad):
  k = pl.kernel(
      _count_body,
      out_type=jax.ShapeDtypeStruct((NW, N * R), jnp.float32),
      mesh=_mesh(),
      compiler_params=_SC_PARAMS,
      scratch_types=[
          pltpu.VMEM((EB,), jnp.int32),
          pltpu.VMEM((EB,), jnp.int32),
          pltpu.VMEM((N * R,), jnp.float32),
      ],
  )
  return k(dst_pad.reshape(NW, EB), type_pad.reshape(NW, EB))


def _wtbl_body(cnt_ref, w_ref):
  s = jnp.sum(cnt_ref[...], axis=0)
  w_ref[...] = 1.0 / jnp.maximum(s, 1.0)


def _wtbl_phase(cnt_parts):
  nrows = (N * R) // 128
  return pl.pallas_call(
      _wtbl_body,
      out_shape=jax.ShapeDtypeStruct((nrows, 128), jnp.float32),
  )(cnt_parts.reshape(NW, nrows, 128))


def _edge_body(src_hbm, dst_hbm, type_hbm, wtbl_hbm,
               hidx_hbm, wedge_hbm, srcb, dstb, typeb, wt):
  wid = _wid()
  pltpu.sync_copy(wtbl_hbm, wt)
  pltpu.sync_copy(src_hbm.at[wid], srcb)
  pltpu.sync_copy(dst_hbm.at[wid], dstb)
  pltpu.sync_copy(type_hbm.at[wid], typeb)

  iota = lax.iota(jnp.int32, L)
  base0 = wid * EB

  def step(i4, _):
    for u in range(4):
      i = i4 * 4 + u
      sl = pl.ds(i * L, L)
      s = srcb[sl]
      t = typeb[sl]
      d = dstb[sl]
      srcb[sl] = t * N + s
      w16 = plsc.load_gather(wt, [d * R + t])
      valid = (base0 + i * L + iota) < E
      w16 = jnp.where(valid, w16, 0.0)
      typeb[sl] = plsc.bitcast(w16, jnp.int32)
    return 0
  lax.fori_loop(0, EB // L // 4, step, 0)

  pltpu.sync_copy(srcb, hidx_hbm.at[wid])
  pltpu.sync_copy(typeb, wedge_hbm.at[wid])


def _edge_phase(src_pad, dst_pad, type_pad, wtbl):
  k = pl.kernel(
      _edge_body,
      out_type=(jax.ShapeDtypeStruct((NW, EB), jnp.int32),
                jax.ShapeDtypeStruct((NW, EB), jnp.int32)),
      mesh=_mesh(),
      compiler_params=_SC_PARAMS,
      scratch_types=[
          pltpu.VMEM((EB,), jnp.int32),
          pltpu.VMEM((EB,), jnp.int32),
          pltpu.VMEM((EB,), jnp.int32),
          pltpu.VMEM((N * R,), jnp.float32),
      ],
  )
  hidx, wedge_i32 = k(src_pad.reshape(NW, EB), dst_pad.reshape(NW, EB),
                      type_pad.reshape(NW, EB), wtbl.reshape(N * R))
  packed = jnp.stack([hidx.reshape(TOT_BLK, B),
                      dst_pad.reshape(TOT_BLK, B),
                      wedge_i32.reshape(TOT_BLK, B)], axis=1)
  return packed


def _msg_body(h_hbm, pk_hbm, out_hbm,
              buf0, buf1, x0, x1, out_sh, rs0, rs1, xs0, xs1):
  c = lax.axis_index("c")
  s = lax.axis_index("s")
  nblk = jnp.where(c == 0, K0, K1)
  base = s * (2 * NBLK) + c * K0
  bufs = (buf0, buf1)
  rsems = (rs0, rs1)
  xbufs = (x0, x1)
  xsems = (xs0, xs1)

  def zbuf(k4, _):
    for u in range(4):
      for cc in range(D // L):
        buf0[k4 * 4 + u, pl.ds(cc * L, L)] = jnp.zeros((L,), jnp.float32)
    return 0
  lax.fori_loop(0, B // 4, zbuf, 0)

  def zrow(g, _):
    b = s + g * NS
    @pl.when(b < NRB)
    def _():
      pltpu.sync_copy(buf0, out_sh.at[pl.ds(b * B, B)])
    return 0
  lax.fori_loop(0, -(-NRB // NS), zrow, 0)
  plsc.subcore_barrier()

  def xload(j, p):
    return pltpu.make_async_copy(pk_hbm.at[base + j], xbufs[p], xsems[p])

  def rgather(j, p):
    return pltpu.make_async_copy(
        h_hbm.at[xbufs[p].at[0]], bufs[p], rsems[p])

  xload(0, 0).start()
  xload(1, 1).start()
  xload(0, 0).wait()
  rgather(0, 0).start()

  def pair(g, _):
    for p in (0, 1):
      j = g * 2 + p

      @pl.when(j < nblk)
      def _():
        rgather(j, p).wait()

        @pl.when(j + 1 < nblk)
        def _():
          xload(j + 1, 1 - p).wait()
          rgather(j + 1, 1 - p).start()

        bufp = bufs[p]
        xp = xbufs[p]

        def scale(k4, _):
          for u in range(4):
            k = k4 * 4 + u
            wbits = plsc.load_gather(xp, [jnp.full((L,), 2, jnp.int32),
                                          jnp.full((L,), k, jnp.int32)])
            w16 = plsc.bitcast(wbits, jnp.float32)
            for cc in range(D // L):
              sl = pl.ds(cc * L, L)
              bufp[k, sl] = bufp[k, sl] * w16
          return 0
        lax.fori_loop(0, B // 4, scale, 0)

        pltpu.sync_copy(bufp, out_sh.at[xp.at[1]], add=True)

        @pl.when(j + 2 < nblk)
        def _():
          xload(j + 2, p).start()
    return 0
  lax.fori_loop(0, (max(K0, K1) + 1) // 2, pair, 0)

  plsc.subcore_barrier()

  def wrow(g, _):
    b = s + g * NS
    @pl.when(b < NRB)
    def _():
      rs = pl.ds(b * B, B)
      pltpu.sync_copy(out_sh.at[rs], out_hbm.at[c, rs])
    return 0
  lax.fori_loop(0, -(-NRB // NS), wrow, 0)


def _msg_phase(h_flat, packed):
  k = pl.kernel(
      _msg_body,
      out_type=jax.ShapeDtypeStruct((NC, N_PAD, D), jnp.float32),
      mesh=_mesh(),
      compiler_params=_SC_PARAMS,
      scratch_types=[
          pltpu.VMEM((B, D), jnp.float32),
          pltpu.VMEM((B, D), jnp.float32),
          pltpu.VMEM((3, B), jnp.int32),
          pltpu.VMEM((3, B), jnp.int32),
          pltpu.VMEM_SHARED((N_PAD, D), jnp.float32),
          pltpu.SemaphoreType.DMA,
          pltpu.SemaphoreType.DMA,
          pltpu.SemaphoreType.DMA,
          pltpu.SemaphoreType.DMA,
      ],
  )
  return k(h_flat, packed)[:, :N, :]


def _dense_phase(xa, xb, w, root, b, relu_in):
  blk = 1000
  nrel = w.shape[0]
  dout = w.shape[2]

  def body(*refs):
    if relu_in:
      xa_ref, xb_ref, w_ref, root_ref, b_ref, h_ref, init_ref = refs
      x = jax.nn.relu(xa_ref[0] + xa_ref[1] + xb_ref[...])
    else:
      xa_ref, w_ref, root_ref, b_ref, h_ref, init_ref = refs
      x = xa_ref[...]
    for r in range(nrel):
      h_ref[r] = jnp.dot(x, w_ref[r], preferred_element_type=jnp.float32)
    init_ref[...] = (
        jnp.dot(x, root_ref[...], preferred_element_type=jnp.float32)
        + b_ref[...])

  if relu_in:
    x_specs = [pl.BlockSpec((NC, blk, D), lambda i: (0, i, 0)),
               pl.BlockSpec((blk, D), lambda i: (i, 0))]
    x_args = [xa, xb]
  else:
    x_specs = [pl.BlockSpec((blk, D), lambda i: (i, 0))]
    x_args = [xa]

  h, init = pl.pallas_call(
      body,
      grid=(N // blk,),
      in_specs=x_specs + [
          pl.BlockSpec((nrel, D, dout), lambda i: (0, 0, 0)),
          pl.BlockSpec((D, dout), lambda i: (0, 0)),
          pl.BlockSpec((1, dout), lambda i: (0, 0)),
      ],
      out_specs=[
          pl.BlockSpec((nrel, blk, dout), lambda i: (0, i, 0)),
          pl.BlockSpec((blk, dout), lambda i: (i, 0)),
      ],
      out_shape=[
          jax.ShapeDtypeStruct((nrel, N, dout), jnp.float32),
          jax.ShapeDtypeStruct((N, dout), jnp.float32),
      ],
  )(*x_args, w, root, b.reshape(1, dout))
  return h.reshape(nrel * N, dout), init


def _final_body(p_ref, init_ref, cw_ref, cb_ref, o_ref):
  h = jax.nn.relu(p_ref[0] + p_ref[1] + init_ref[...])
  o_ref[...] = (
      jnp.dot(h, cw_ref[...], preferred_element_type=jnp.float32)
      + cb_ref[...])


def _final_phase(parts, init, cls_w, cls_b):
  blk = 1000
  out = pl.pallas_call(
      _final_body,
      grid=(N // blk,),
      in_specs=[
          pl.BlockSpec((NC, blk, D), lambda i: (0, i, 0)),
          pl.BlockSpec((blk, D), lambda i: (i, 0)),
          pl.BlockSpec((D, 1), lambda i: (0, 0)),
          pl.BlockSpec((1, 1), lambda i: (0, 0)),
      ],
      out_specs=pl.BlockSpec((blk, 1), lambda i: (i, 0)),
      out_shape=jax.ShapeDtypeStruct((N, 1), jnp.float32),
  )(parts, init, cls_w, cls_b.reshape(1, 1))
  return out.reshape(N)


@jax.jit
def kernel(x, edge_index, edge_type, W1, root1, b1, W2, root2, b2,
           cls_w, cls_b):
  pad = E_PAD - E
  src_pad = jnp.concatenate(
      [edge_index[0], jnp.zeros((pad,), jnp.int32)])
  dst_pad = jnp.concatenate(
      [edge_index[1], jnp.zeros((pad,), jnp.int32)])
  type_pad = jnp.concatenate([edge_type, jnp.zeros((pad,), jnp.int32)])

  cnt_parts = _count_phase(dst_pad, type_pad)
  wtbl = _wtbl_phase(cnt_parts)
  packed = _edge_phase(src_pad, dst_pad, type_pad, wtbl)

  h1, init1 = _dense_phase(x, None, W1, root1, b1, relu_in=False)
  p1 = _msg_phase(h1, packed)
  h2, init2 = _dense_phase(p1, init1, W2, root2, b2, relu_in=True)
  p2 = _msg_phase(h2, packed)
  return _final_phase(p2, init2, cls_w, cls_b)


# speedup vs baseline: 1.2689x; 1.0389x over previous
import functools

import jax
import jax.numpy as jnp
from jax import lax
from jax.experimental import pallas as pl
from jax.experimental.pallas import tpu as pltpu
from jax.experimental.pallas import tpu_sc as plsc

N = 10000
D = 128
R = 8
E = 320000

NC = 2
NS = 16
NW = NC * NS
L = 16

B = 128
NBLK = -(-E // (NW * B))
EB = NBLK * B
E_PAD = NW * EB

N_PAD = NBLK * B
NRB = N_PAD // B

K0 = 108
K1 = 2 * NBLK - K0
TOT_BLK = NS * 2 * NBLK

_mesh = functools.partial(
    plsc.VectorSubcoreMesh, core_axis_name="c", subcore_axis_name="s")
_SC_PARAMS = pltpu.CompilerParams(needs_layout_passes=False)


def _wid():
  return lax.axis_index("s") * NC + lax.axis_index("c")


def _count_body(dst_hbm, type_hbm, cnt_hbm, dstb, typeb, tbl):
  wid = _wid()

  def zero_step(i, _):
    tbl[pl.ds(i * L, L)] = jnp.zeros((L,), jnp.float32)
    return 0
  lax.fori_loop(0, (N * R) // L, zero_step, 0)

  pltpu.sync_copy(dst_hbm.at[wid], dstb)
  pltpu.sync_copy(type_hbm.at[wid], typeb)

  ones = jnp.ones((L,), jnp.float32)
  iota = lax.iota(jnp.int32, L)
  base0 = wid * EB

  def step(i, _):
    d = dstb[pl.ds(i * L, L)]
    t = typeb[pl.ds(i * L, L)]
    idx = d * R + t
    valid = (base0 + i * L + iota) < E
    plsc.addupdate_scatter(tbl, [idx], ones, mask=valid)
    return 0
  lax.fori_loop(0, EB // L, step, 0)

  pltpu.sync_copy(tbl, cnt_hbm.at[wid])


def _count_phase(dst_pad, type_pad):
  k = pl.kernel(
      _count_body,
      out_type=jax.ShapeDtypeStruct((NW, N * R), jnp.float32),
      mesh=_mesh(),
      compiler_params=_SC_PARAMS,
      scratch_types=[
          pltpu.VMEM((EB,), jnp.int32),
          pltpu.VMEM((EB,), jnp.int32),
          pltpu.VMEM((N * R,), jnp.float32),
      ],
  )
  return k(dst_pad.reshape(NW, EB), type_pad.reshape(NW, EB))


def _wtbl_body(cnt_ref, w_ref):
  s = jnp.sum(cnt_ref[...], axis=0)
  w_ref[...] = 1.0 / jnp.maximum(s, 1.0)


def _wtbl_phase(cnt_parts):
  nrows = (N * R) // 128
  return pl.pallas_call(
      _wtbl_body,
      out_shape=jax.ShapeDtypeStruct((nrows, 128), jnp.float32),
  )(cnt_parts.reshape(NW, nrows, 128))


def _edge_body(src_hbm, dst_hbm, type_hbm, wtbl_hbm,
               hidx_hbm, wedge_hbm, srcb, dstb, typeb, wt):
  wid = _wid()
  pltpu.sync_copy(wtbl_hbm, wt)
  pltpu.sync_copy(src_hbm.at[wid], srcb)
  pltpu.sync_copy(dst_hbm.at[wid], dstb)
  pltpu.sync_copy(type_hbm.at[wid], typeb)

  iota = lax.iota(jnp.int32, L)
  base0 = wid * EB

  def step(i4, _):
    for u in range(4):
      i = i4 * 4 + u
      sl = pl.ds(i * L, L)
      s = srcb[sl]
      t = typeb[sl]
      d = dstb[sl]
      srcb[sl] = t * N + s
      w16 = plsc.load_gather(wt, [d * R + t])
      valid = (base0 + i * L + iota) < E
      w16 = jnp.where(valid, w16, 0.0)
      typeb[sl] = plsc.bitcast(w16, jnp.int32)
    return 0
  lax.fori_loop(0, EB // L // 4, step, 0)

  pltpu.sync_copy(srcb, hidx_hbm.at[wid])
  pltpu.sync_copy(typeb, wedge_hbm.at[wid])


def _edge_phase(src_pad, dst_pad, type_pad, wtbl):
  k = pl.kernel(
      _edge_body,
      out_type=(jax.ShapeDtypeStruct((NW, EB), jnp.int32),
                jax.ShapeDtypeStruct((NW, EB), jnp.int32)),
      mesh=_mesh(),
      compiler_params=_SC_PARAMS,
      scratch_types=[
          pltpu.VMEM((EB,), jnp.int32),
          pltpu.VMEM((EB,), jnp.int32),
          pltpu.VMEM((EB,), jnp.int32),
          pltpu.VMEM((N * R,), jnp.float32),
      ],
  )
  hidx, wedge_i32 = k(src_pad.reshape(NW, EB), dst_pad.reshape(NW, EB),
                      type_pad.reshape(NW, EB), wtbl.reshape(N * R))
  packed = jnp.stack([hidx.reshape(TOT_BLK, B),
                      dst_pad.reshape(TOT_BLK, B),
                      wedge_i32.reshape(TOT_BLK, B)], axis=1)
  return packed


def _msg_body(h_hbm, pk_hbm, out_hbm,
              buf0, buf1, x0, x1, out_sh, rs0, rs1, xs0, xs1):
  c = lax.axis_index("c")
  s = lax.axis_index("s")
  nblk = jnp.where(c == 0, K0, K1)
  base = s * (2 * NBLK) + c * K0
  bufs = (buf0, buf1)
  rsems = (rs0, rs1)
  xbufs = (x0, x1)
  xsems = (xs0, xs1)

  def zbuf(k4, _):
    for u in range(4):
      for cc in range(D // L):
        buf0[k4 * 4 + u, pl.ds(cc * L, L)] = jnp.zeros((L,), jnp.float32)
    return 0
  lax.fori_loop(0, B // 4, zbuf, 0)

  def zrow(g, _):
    b = s + g * NS
    @pl.when(b < NRB)
    def _():
      pltpu.sync_copy(buf0, out_sh.at[pl.ds(b * B, B)])
    return 0
  lax.fori_loop(0, -(-NRB // NS), zrow, 0)
  plsc.subcore_barrier()

  def xload(j, p):
    return pltpu.make_async_copy(pk_hbm.at[base + j], xbufs[p], xsems[p])

  def rgather(j, p):
    return pltpu.make_async_copy(
        h_hbm.at[xbufs[p].at[0]], bufs[p], rsems[p])

  xload(0, 0).start()
  xload(1, 1).start()
  xload(0, 0).wait()
  rgather(0, 0).start()

  def pair(g, _):
    for p in (0, 1):
      j = g * 2 + p

      @pl.when(j < nblk)
      def _():
        rgather(j, p).wait()

        @pl.when(j + 1 < nblk)
        def _():
          xload(j + 1, 1 - p).wait()
          rgather(j + 1, 1 - p).start()

        bufp = bufs[p]
        xp = xbufs[p]

        def scale(k4, _):
          for u in range(4):
            k = k4 * 4 + u
            wbits = plsc.load_gather(xp, [jnp.full((L,), 2, jnp.int32),
                                          jnp.full((L,), k, jnp.int32)])
            w16 = plsc.bitcast(wbits, jnp.float32)
            for cc in range(D // L):
              sl = pl.ds(cc * L, L)
              bufp[k, sl] = bufp[k, sl] * w16
          return 0
        lax.fori_loop(0, B // 4, scale, 0)

        pltpu.sync_copy(bufp, out_sh.at[xp.at[1]], add=True)

        @pl.when(j + 2 < nblk)
        def _():
          xload(j + 2, p).start()
    return 0
  lax.fori_loop(0, (max(K0, K1) + 1) // 2, pair, 0)

  plsc.subcore_barrier()

  def wrow(g, _):
    b = s + g * NS
    @pl.when(b < NRB)
    def _():
      rs = pl.ds(b * B, B)
      pltpu.sync_copy(out_sh.at[rs], out_hbm.at[c, rs])
    return 0
  lax.fori_loop(0, -(-NRB // NS), wrow, 0)


def _msg_phase(h_flat, packed):
  k = pl.kernel(
      _msg_body,
      out_type=jax.ShapeDtypeStruct((NC, N_PAD, D), jnp.float32),
      mesh=_mesh(),
      compiler_params=_SC_PARAMS,
      scratch_types=[
          pltpu.VMEM((B, D), jnp.float32),
          pltpu.VMEM((B, D), jnp.float32),
          pltpu.VMEM((3, B), jnp.int32),
          pltpu.VMEM((3, B), jnp.int32),
          pltpu.VMEM_SHARED((N_PAD, D), jnp.float32),
          pltpu.SemaphoreType.DMA,
          pltpu.SemaphoreType.DMA,
          pltpu.SemaphoreType.DMA,
          pltpu.SemaphoreType.DMA,
      ],
  )
  return k(h_flat, packed)[:, :N, :]


def _dense_phase(xa, xb, w, root, b, relu_in):
  blk = 1000
  nrel = w.shape[0]
  dout = w.shape[2]

  def body(*refs):
    if relu_in:
      xa_ref, xb_ref, w_ref, root_ref, b_ref, h_ref, init_ref = refs
      x = jax.nn.relu(xa_ref[0] + xa_ref[1] + xb_ref[...])
    else:
      xa_ref, w_ref, root_ref, b_ref, h_ref, init_ref = refs
      x = xa_ref[...]
    for r in range(nrel):
      h_ref[r] = jnp.dot(x, w_ref[r], preferred_element_type=jnp.float32)
    init_ref[...] = (
        jnp.dot(x, root_ref[...], preferred_element_type=jnp.float32)
        + b_ref[...])

  if relu_in:
    x_specs = [pl.BlockSpec((NC, blk, D), lambda i: (0, i, 0)),
               pl.BlockSpec((blk, D), lambda i: (i, 0))]
    x_args = [xa, xb]
  else:
    x_specs = [pl.BlockSpec((blk, D), lambda i: (i, 0))]
    x_args = [xa]

  h, init = pl.pallas_call(
      body,
      grid=(N // blk,),
      in_specs=x_specs + [
          pl.BlockSpec((nrel, D, dout), lambda i: (0, 0, 0)),
          pl.BlockSpec((D, dout), lambda i: (0, 0)),
          pl.BlockSpec((1, dout), lambda i: (0, 0)),
      ],
      out_specs=[
          pl.BlockSpec((nrel, blk, dout), lambda i: (0, i, 0)),
          pl.BlockSpec((blk, dout), lambda i: (i, 0)),
      ],
      out_shape=[
          jax.ShapeDtypeStruct((nrel, N, dout), jnp.float32),
          jax.ShapeDtypeStruct((N, dout), jnp.float32),
      ],
  )(*x_args, w, root, b.reshape(1, dout))
  return h.reshape(nrel * N, dout), init


def _final_body(p_ref, init_ref, cw_ref, cb_ref, o_ref):
  h = jax.nn.relu(p_ref[0] + p_ref[1] + init_ref[...])
  o_ref[...] = (
      jnp.dot(h, cw_ref[...], preferred_element_type=jnp.float32)
      + cb_ref[...])


def _final_phase(parts, init, cls_w, cls_b):
  blk = 1000
  out = pl.pallas_call(
      _final_body,
      grid=(N // blk,),
      in_specs=[
          pl.BlockSpec((NC, blk, D), lambda i: (0, i, 0)),
          pl.BlockSpec((blk, D), lambda i: (i, 0)),
          pl.BlockSpec((D, 1), lambda i: (0, 0)),
          pl.BlockSpec((1, 1), lambda i: (0, 0)),
      ],
      out_specs=pl.BlockSpec((blk, 1), lambda i: (i, 0)),
      out_shape=jax.ShapeDtypeStruct((N, 1), jnp.float32),
  )(parts, init, cls_w, cls_b.reshape(1, 1))
  return out.reshape(N)


@jax.jit
def kernel(x, edge_index, edge_type, W1, root1, b1, W2, root2, b2,
           cls_w, cls_b):
  pad = E_PAD - E
  src_pad = jnp.concatenate(
      [edge_index[0], jnp.zeros((pad,), jnp.int32)])
  dst_pad = jnp.concatenate(
      [edge_index[1], jnp.zeros((pad,), jnp.int32)])
  type_pad = jnp.concatenate([edge_type, jnp.zeros((pad,), jnp.int32)])

  cnt_parts = _count_phase(dst_pad, type_pad)
  wtbl = _wtbl_phase(cnt_parts)
  packed = _edge_phase(src_pad, dst_pad, type_pad, wtbl)

  h1, init1 = _dense_phase(x, None, W1, root1, b1, relu_in=False)
  p1 = _msg_phase(h1, packed)
  h2, init2 = _dense_phase(p1, init1, W2, root2, b2, relu_in=True)
  p2 = _msg_phase(h2, packed)
  return _final_phase(p2, init2, cls_w, cls_b)
